# Initial kernel scaffold; baseline (speedup 1.0000x reference)
import jax
import jax.numpy as jnp
from jax.experimental import pallas as pl


def kernel(x_e, edge_index, rel, rel_all, w_h, w_t, a_h1, a_h2, a_t1, a_t2, r_h_w, r_h_b, r_t_w, r_t_b):
    raise NotImplementedError("write your pallas kernel here")


# speedup vs baseline: 34.5315x; 34.5315x over previous
import jax
import jax.numpy as jnp
from jax import lax
from jax.experimental import pallas as pl
from jax.experimental.pallas import tpu as pltpu
from jax.experimental.pallas import tpu_sc as plsc

N_NODES = 10000
N_EDGES = 320000
HID = 128
N_REL = 1000
R_PAD = 1024

N_TILES = 16
E_PER_TILE = N_EDGES // N_TILES
CE = 4000
N_CHUNKS = E_PER_TILE // CE
CH = 80
G_PER_CHUNK = CE // CH

_BLK = 1000
_NB = N_NODES // _BLK


def _lane_splat(vec, j):
    idx = jnp.full((16, 1), j, jnp.int32)
    dn = lax.GatherDimensionNumbers(
        offset_dims=(), collapsed_slice_dims=(0,), start_index_map=(0,)
    )
    return lax.gather(
        vec, idx, dn, slice_sizes=(1,),
        mode=lax.GatherScatterMode.PROMISE_IN_BOUNDS,
    )


def _tc_body(x_ref, w_ref, u_ref, v_ref, r_ref, rb_ref, m_ref, p_ref, q_ref):
    x = x_ref[...]
    xr = jnp.dot(x, w_ref[0], preferred_element_type=jnp.float32)
    p_ref[0] = jnp.dot(xr, u_ref[0], preferred_element_type=jnp.float32)
    q_ref[0] = jnp.dot(xr, v_ref[0], preferred_element_type=jnp.float32)
    m_ref[0] = (
        jnp.dot(xr, r_ref[0], preferred_element_type=jnp.float32)
        + rb_ref[0]
        + xr
    )


@jax.jit
def _tc_stage(x_e, W, U, V, R, Rb):
    return pl.pallas_call(
        _tc_body,
        grid=(2, _NB),
        in_specs=[
            pl.BlockSpec((_BLK, HID), lambda s, i: (i, 0)),
            pl.BlockSpec((1, HID, HID), lambda s, i: (s, 0, 0)),
            pl.BlockSpec((1, HID, 1), lambda s, i: (s, 0, 0)),
            pl.BlockSpec((1, HID, 1), lambda s, i: (s, 0, 0)),
            pl.BlockSpec((1, HID, HID), lambda s, i: (s, 0, 0)),
            pl.BlockSpec((1, 1, HID), lambda s, i: (s, 0, 0)),
        ],
        out_specs=[
            pl.BlockSpec((1, _BLK, HID), lambda s, i: (s, i, 0)),
            pl.BlockSpec((1, _BLK, 1), lambda s, i: (s, i, 0)),
            pl.BlockSpec((1, _BLK, 1), lambda s, i: (s, i, 0)),
        ],
        out_shape=[
            jax.ShapeDtypeStruct((2, N_NODES, HID), jnp.float32),
            jax.ShapeDtypeStruct((2, N_NODES, 1), jnp.float32),
            jax.ShapeDtypeStruct((2, N_NODES, 1), jnp.float32),
        ],
    )(x_e, W, U, V, R, Rb)


def _sc_body(
    h_hbm, t_hbm, rel_hbm, sa_hbm, sb_hbm, m_hbm, bnd_hbm,
    out_hbm, den_hbm,
    sa_v, sb_v, h_c, t_c, r_c, den_v, tmp_v,
    gm0, sr0, al0, rows0, gm1, sr1, al1, rows1,
    bnd_v, zero_v, acc_sh, sem0, sem1, sems0, sems1,
):
    c = lax.axis_index("c")
    s = lax.axis_index("s")
    base = s * E_PER_TILE
    zero16 = jnp.zeros((16,), jnp.float32)

    pltpu.sync_copy(sa_hbm.at[pl.ds(c * N_NODES, N_NODES)], sa_v)
    pltpu.sync_copy(sb_hbm.at[pl.ds(c * N_NODES, N_NODES)], sb_v)
    pltpu.sync_copy(bnd_hbm, bnd_v)
    bc = bnd_v[c]

    for i in range(8):
        for u in range(8):
            den_v[i, pl.ds(u * 16, 16)] = zero16
    for j in range(16):
        for k in range(8):
            zero_v[j, pl.ds(k * 16, 16)] = zero16

    def _zacc(i, carry):
        pltpu.sync_copy(zero_v, acc_sh.at[pl.ds(s * 64 + i * 16, 16)])
        return carry

    lax.fori_loop(0, 4, _zacc, 0)
    plsc.subcore_barrier()

    def _edge_vec(off):
        hv = h_c[pl.ds(off, 16)]
        tv = t_c[pl.ds(off, 16)]
        rv = r_c[pl.ds(off, 16)]
        av = plsc.load_gather(sa_v, [hv])
        bv = plsc.load_gather(sb_v, [tv])
        e = av + bv
        e = jnp.where(e >= 0.0, e, e * 0.01)
        ex = jnp.exp(e - bc)
        return hv, tv, rv, ex

    def _p1_chunk(ch, carry):
        off = base + ch * CE
        pltpu.sync_copy(h_hbm.at[pl.ds(off, CE)], h_c)
        pltpu.sync_copy(t_hbm.at[pl.ds(off, CE)], t_c)
        pltpu.sync_copy(rel_hbm.at[pl.ds(off, CE)], r_c)

        def _p1_vec(i, carry2):
            _, _, rv, ex = _edge_vec(i * 16)
            plsc.addupdate_scatter(den_v, [rv >> 7, rv & 127], ex)
            return carry2

        lax.fori_loop(0, CE // 16, _p1_vec, 0)
        return carry

    lax.fori_loop(0, N_CHUNKS, _p1_chunk, 0)

    pltpu.sync_copy(den_v, den_hbm.at[c, s])
    plsc.subcore_barrier()
    pltpu.sync_copy(den_hbm.at[c], tmp_v)
    for j in range(8):
        for u in range(8):
            sl = pl.ds(u * 16, 16)
            acc = tmp_v[0, j, sl]
            for k in range(1, 16):
                acc = acc + tmp_v[k, j, sl]
            den_v[j, sl] = acc

    def _fill(goff, gm_b, sr_b, al_b):
        for v in range(CH // 16):
            hv, tv, rv, ex = _edge_vec(goff + v * 16)
            dv = plsc.load_gather(den_v, [rv >> 7, rv & 127])
            al_b[pl.ds(v * 16, 16)] = ex / (dv + 1e-16)
            sr_b[pl.ds(v * 16, 16)] = rv
            gm_b[pl.ds(v * 16, 16)] = hv + c * (tv + N_NODES - hv)

    def _scale(rows_b, al_b):
        for v in range(CH // 16):
            a16 = al_b[pl.ds(v * 16, 16)]
            for j in range(16):
                aj = _lane_splat(a16, j)
                for k in range(8):
                    sl = pl.ds(k * 16, 16)
                    rows_b[v * 16 + j, sl] = rows_b[v * 16 + j, sl] * aj

    def _wait_gather(rows_b, sem_b):
        pltpu.make_async_copy(m_hbm.at[pl.ds(0, CH)], rows_b, sem_b).wait()

    def _drain_scatter(rows_b, sem_b):
        pltpu.make_async_copy(m_hbm.at[pl.ds(0, CH)], rows_b, sem_b).wait()

    def _p2_chunk(ch, carry):
        off = base + ch * CE
        pltpu.sync_copy(h_hbm.at[pl.ds(off, CE)], h_c)
        pltpu.sync_copy(t_hbm.at[pl.ds(off, CE)], t_c)
        pltpu.sync_copy(rel_hbm.at[pl.ds(off, CE)], r_c)

        @pl.when(ch > 0)
        def _():
            _drain_scatter(rows0, sems0)
            _drain_scatter(rows1, sems1)

        _fill(0, gm0, sr0, al0)
        pltpu.async_copy(m_hbm.at[gm0], rows0, sem0)

        def _pair(p, carry2):
            @pl.when(p > 0)
            def _():
                _drain_scatter(rows1, sems1)

            _fill((2 * p + 1) * CH, gm1, sr1, al1)
            pltpu.async_copy(m_hbm.at[gm1], rows1, sem1)
            _wait_gather(rows0, sem0)
            _scale(rows0, al0)
            pltpu.async_copy(rows0, acc_sh.at[sr0], sems0, add=True)

            @pl.when(p < G_PER_CHUNK // 2 - 1)
            def _():
                _drain_scatter(rows0, sems0)
                _fill((2 * p + 2) * CH, gm0, sr0, al0)
                pltpu.async_copy(m_hbm.at[gm0], rows0, sem0)

            _wait_gather(rows1, sem1)
            _scale(rows1, al1)
            pltpu.async_copy(rows1, acc_sh.at[sr1], sems1, add=True)
            return carry2

        lax.fori_loop(0, G_PER_CHUNK // 2, _pair, 0)
        return carry

    lax.fori_loop(0, N_CHUNKS, _p2_chunk, 0)
    _drain_scatter(rows0, sems0)
    _drain_scatter(rows1, sems1)

    plsc.subcore_barrier()

    def _wb(i, carry):
        r0 = s * 64 + i * 16
        pltpu.sync_copy(acc_sh.at[pl.ds(r0, 16)], out_hbm.at[c, pl.ds(r0, 16)])
        return carry

    lax.fori_loop(0, 4, _wb, 0)


@jax.jit
def _sc_stage(h_idx, t_idx, rel, sa, sb, m_flat, bnd):
    mesh = plsc.VectorSubcoreMesh(
        core_axis_name="c", subcore_axis_name="s",
        num_cores=2, num_subcores=N_TILES,
    )
    fn = pl.kernel(
        _sc_body,
        out_type=[
            jax.ShapeDtypeStruct((2, R_PAD, HID), jnp.float32),
            jax.ShapeDtypeStruct((2, N_TILES, 8, 128), jnp.float32),
        ],
        mesh=mesh,
        compiler_params=pltpu.CompilerParams(needs_layout_passes=False),
        scratch_types=[
            pltpu.VMEM((N_NODES,), jnp.float32),
            pltpu.VMEM((N_NODES,), jnp.float32),
            pltpu.VMEM((CE,), jnp.int32),
            pltpu.VMEM((CE,), jnp.int32),
            pltpu.VMEM((CE,), jnp.int32),
            pltpu.VMEM((8, 128), jnp.float32),
            pltpu.VMEM((N_TILES, 8, 128), jnp.float32),
            pltpu.VMEM((CH,), jnp.int32),
            pltpu.VMEM((CH,), jnp.int32),
            pltpu.VMEM((CH,), jnp.float32),
            pltpu.VMEM((CH, HID), jnp.float32),
            pltpu.VMEM((CH,), jnp.int32),
            pltpu.VMEM((CH,), jnp.int32),
            pltpu.VMEM((CH,), jnp.float32),
            pltpu.VMEM((CH, HID), jnp.float32),
            pltpu.VMEM((2, 16), jnp.float32),
            pltpu.VMEM((16, HID), jnp.float32),
            pltpu.VMEM_SHARED((R_PAD, HID), jnp.float32),
            pltpu.SemaphoreType.DMA,
            pltpu.SemaphoreType.DMA,
            pltpu.SemaphoreType.DMA,
            pltpu.SemaphoreType.DMA,
        ],
    )
    out, _ = fn(h_idx, t_idx, rel, sa, sb, m_flat, bnd)
    return out


def kernel(x_e, edge_index, rel, rel_all, w_h, w_t, a_h1, a_h2, a_t1, a_t2,
           r_h_w, r_h_b, r_t_w, r_t_b):
    W = jnp.stack([w_h, w_t])
    U = jnp.stack([a_h1, a_h2])[:, :, None]
    V = jnp.stack([a_t1, a_t2])[:, :, None]
    R = jnp.stack([r_h_w, r_t_w])
    Rb = jnp.stack([r_h_b, r_t_b])[:, None, :]

    M, P, Q = _tc_stage(x_e, W, U, V, R, Rb)
    p0 = P[0, :, 0]
    p1 = P[1, :, 0]
    q0 = Q[0, :, 0]
    q1 = Q[1, :, 0]
    sa = jnp.concatenate([p0, q0])
    sb = jnp.concatenate([p1, q1])
    c1 = jax.nn.leaky_relu(jnp.max(p0) + jnp.max(p1), 0.01)
    c2 = jax.nn.leaky_relu(jnp.max(q0) + jnp.max(q1), 0.01)
    bnd = jnp.stack([jnp.full((16,), c1), jnp.full((16,), c2)])
    m_flat = M.reshape(2 * N_NODES, HID)

    out2 = _sc_stage(edge_index[0], edge_index[1], rel, sa, sb, m_flat, bnd)
    return out2[0, :N_REL] + out2[1, :N_REL]
